# baseline (speedup 1.0000x reference)
import functools

import jax
import jax.numpy as jnp
from jax import lax
from jax.experimental import pallas as pl
from jax.experimental.pallas import tpu as pltpu
from jax.experimental.pallas import tpu_sc as plsc

NC = 2
NS = 16
NW = NC * NS
CH = 128
JUNK = 128


def _sc_mesh():
  return plsc.VectorSubcoreMesh(core_axis_name="c", subcore_axis_name="s")


STRIPE = 640
SUB = 80
DEGW = 16


def _make_deg_kernel(n, e_pad):
  full = e_pad // CH
  per = full // NW
  assert per * NW == full and per % 8 == 0
  n_sub = STRIPE // SUB
  mesh = _sc_mesh()

  @functools.partial(
      pl.kernel,
      out_type=jax.ShapeDtypeStruct((NC, n, DEGW), jnp.float32),
      mesh=mesh,
      scratch_types=[
          pltpu.VMEM((per, CH), jnp.int32),
          pltpu.VMEM((CH, DEGW), jnp.float32),
          pltpu.VMEM_SHARED((n + JUNK, DEGW), jnp.float32),
      ],
  )
  def deg_kernel(dst2_hbm, ones_hbm, out_hbm, didx_v, ones_v, acc_sh):
    c = lax.axis_index("c")
    s = lax.axis_index("s")
    w = s * NC + c
    r0 = s * STRIPE

    pltpu.sync_copy(dst2_hbm.at[pl.ds(w * per, per)], didx_v)

    pltpu.sync_copy(ones_hbm, ones_v)
    for q in range(n_sub):
      @pl.when(r0 + q * SUB < n)
      def _init():
        pltpu.sync_copy(
            ones_v.at[pl.ds(0, SUB)], acc_sh.at[pl.ds(r0 + q * SUB, SUB)])
    plsc.subcore_barrier()

    for j in range(per):
      pltpu.sync_copy(
          ones_v.at[pl.ds(0, CH)], acc_sh.at[didx_v.at[j]], add=True)

    plsc.subcore_barrier()
    for q in range(n_sub):
      @pl.when(r0 + q * SUB < n)
      def _out():
        pltpu.sync_copy(
            acc_sh.at[pl.ds(r0 + q * SUB, SUB)],
            out_hbm.at[c, pl.ds(r0 + q * SUB, SUB)])

  return deg_kernel


NBUF = 2


def _make_scatter_kernel(n, e_pad, d):
  full = e_pad // CH
  per = full // NW
  assert per * NW == full and per % 8 == 0 and per % NBUF == 0
  n_sub = STRIPE // SUB
  mesh = _sc_mesh()

  @functools.partial(
      pl.kernel,
      out_type=jax.ShapeDtypeStruct((NC, n, d), jnp.float32),
      mesh=mesh,
      scratch_types=[
          pltpu.VMEM((1, CH), jnp.int32),
          pltpu.VMEM((1, CH), jnp.int32),
          pltpu.VMEM((CH, d), jnp.float32),
          pltpu.VMEM_SHARED((n + JUNK, d), jnp.float32),
      ],
  )
  def scatter_kernel(src_hbm, dst_hbm, y_hbm, zeros_hbm, out_hbm,
                     sidx_v, didx_v, rows_v, acc_sh):
    c = lax.axis_index("c")
    s = lax.axis_index("s")
    w = s * NC + c
    r0 = s * STRIPE

    pltpu.sync_copy(zeros_hbm, rows_v.at[pl.ds(0, SUB)])
    for q in range(n_sub):
      @pl.when(r0 + q * SUB < n)
      def _init():
        pltpu.sync_copy(
            rows_v.at[pl.ds(0, SUB)],
            acc_sh.at[pl.ds(r0 + q * SUB, SUB)])
    plsc.subcore_barrier()

    def body(j, carry):
      base = (w * per + j) * CH
      pltpu.sync_copy(src_hbm.at[pl.ds(base, CH)], sidx_v.at[0])
      pltpu.sync_copy(dst_hbm.at[pl.ds(base, CH)], didx_v.at[0])
      pltpu.sync_copy(y_hbm.at[sidx_v.at[0]], rows_v)
      pltpu.sync_copy(rows_v, acc_sh.at[didx_v.at[0]], add=True)
      return carry

    lax.fori_loop(0, per, body, 0)

    plsc.subcore_barrier()
    for q in range(n_sub):
      @pl.when(r0 + q * SUB < n)
      def _out():
        pltpu.sync_copy(
            acc_sh.at[pl.ds(r0 + q * SUB, SUB)],
            out_hbm.at[c, pl.ds(r0 + q * SUB, SUB)])

  return scatter_kernel


ROW_BLK = 1000


def _tc_first(dcol_ref, x_ref, w_ref, y_ref):
  dinv = lax.rsqrt(dcol_ref[...])
  xw = jnp.dot(x_ref[...], w_ref[...], preferred_element_type=jnp.float32)
  y_ref[...] = xw * dinv


def _tc_mid(dcol_ref, s_ref, y_ref, b_ref, w_ref, o_ref):
  dinv = lax.rsqrt(dcol_ref[...])
  tot = s_ref[0] + s_ref[1] + y_ref[...]
  h = jnp.maximum(tot * dinv + b_ref[...], 0.0)
  o_ref[...] = jnp.dot(
      h, w_ref[...], preferred_element_type=jnp.float32) * dinv


def _tc_last(dcol_ref, s_ref, y_ref, b_ref, w_ref, bl_ref, o_ref):
  dinv = lax.rsqrt(dcol_ref[...])
  tot = s_ref[0] + s_ref[1] + y_ref[...]
  h = jnp.maximum(tot * dinv + b_ref[...], 0.0)
  o_ref[...] = jnp.dot(
      h, w_ref[...], preferred_element_type=jnp.float32) + bl_ref[...]


def _dcol_spec():
  return pl.BlockSpec((ROW_BLK, 1), lambda i: (i, 0))


def _row_spec(d):
  return pl.BlockSpec((ROW_BLK, d), lambda i: (i, 0))


def _full_spec(shape):
  nd = len(shape)
  return pl.BlockSpec(shape, lambda i: (0,) * nd)


def _tc1(dcol, x, w1, n, d):
  return pl.pallas_call(
      _tc_first,
      grid=(n // ROW_BLK,),
      in_specs=[_dcol_spec(), _row_spec(d), _full_spec((d, d))],
      out_specs=_row_spec(d),
      out_shape=jax.ShapeDtypeStruct((n, d), jnp.float32),
  )(dcol, x, w1)


def _tc2(dcol, s1, y1, b1r, w2, n, d):
  return pl.pallas_call(
      _tc_mid,
      grid=(n // ROW_BLK,),
      in_specs=[
          _dcol_spec(),
          pl.BlockSpec((NC, ROW_BLK, d), lambda i: (0, i, 0)),
          _row_spec(d),
          _full_spec((1, d)),
          _full_spec((d, d)),
      ],
      out_specs=_row_spec(d),
      out_shape=jax.ShapeDtypeStruct((n, d), jnp.float32),
  )(dcol, s1, y1, b1r, w2)


def _tc3(dcol, s2, y2, b2r, wl, blr, n, d):
  return pl.pallas_call(
      _tc_last,
      grid=(n // ROW_BLK,),
      in_specs=[
          _dcol_spec(),
          pl.BlockSpec((NC, ROW_BLK, d), lambda i: (0, i, 0)),
          _row_spec(d),
          _full_spec((1, d)),
          _full_spec((d, d)),
          _full_spec((1, d)),
      ],
      out_specs=_row_spec(d),
      out_shape=jax.ShapeDtypeStruct((n, d), jnp.float32),
  )(dcol, s2, y2, b2r, wl, blr)


def kernel(x, edge_index, W1, b1, W2, b2, Wl, bl):
  n, d = x.shape
  e = edge_index.shape[1]

  e_pad = e + (-e) % (CH * 8 * NW)
  pad = e_pad - e
  src1 = jnp.concatenate([edge_index[0], jnp.zeros((pad,), jnp.int32)])
  pad_dst = n + jnp.arange(pad, dtype=jnp.int32) % JUNK
  dst1 = jnp.concatenate([edge_index[1], pad_dst])
  dst = dst1.reshape(e_pad // CH, CH)
  ones_col = jnp.ones((CH, DEGW), jnp.float32)
  zeros_blk = jnp.zeros((SUB, d), jnp.float32)
  b1r = b1.reshape(1, d)
  b2r = b2.reshape(1, d)
  blr = bl.reshape(1, d)

  deg_kernel = _make_deg_kernel(n, e_pad)
  scatter_kernel = _make_scatter_kernel(n, e_pad, d)

  deg2 = deg_kernel(dst, ones_col)
  dcol = deg2[0, :, :1] + deg2[1, :, :1] - 1.0

  y1 = _tc1(dcol, x, W1, n, d)
  s1 = scatter_kernel(src1, dst1, y1, zeros_blk)
  y2 = _tc2(dcol, s1, y1, b1r, W2, n, d)
  s2 = scatter_kernel(src1, dst1, y2, zeros_blk)
  return _tc3(dcol, s2, y2, b2r, Wl, blr, n, d)


# speedup vs baseline: 2.1150x; 2.1150x over previous
import functools

import jax
import jax.numpy as jnp
from jax import lax
from jax.experimental import pallas as pl
from jax.experimental.pallas import tpu as pltpu
from jax.experimental.pallas import tpu_sc as plsc

NC = 2
NS = 16
NW = NC * NS
CH = 128
JUNK = 128


def _sc_mesh():
  return plsc.VectorSubcoreMesh(core_axis_name="c", subcore_axis_name="s")


STRIPE = 640
SUB = 80
DEGW = 16


def _make_deg_kernel(n, e_pad):
  full = e_pad // CH
  per = full // NW
  assert per * NW == full and per % 8 == 0
  n_sub = STRIPE // SUB
  mesh = _sc_mesh()

  @functools.partial(
      pl.kernel,
      out_type=jax.ShapeDtypeStruct((NC, n, DEGW), jnp.float32),
      mesh=mesh,
      scratch_types=[
          pltpu.VMEM((per, CH), jnp.int32),
          pltpu.VMEM((CH, DEGW), jnp.float32),
          pltpu.VMEM_SHARED((n + JUNK, DEGW), jnp.float32),
      ],
  )
  def deg_kernel(dst2_hbm, ones_hbm, out_hbm, didx_v, ones_v, acc_sh):
    c = lax.axis_index("c")
    s = lax.axis_index("s")
    w = s * NC + c
    r0 = s * STRIPE

    pltpu.sync_copy(dst2_hbm.at[pl.ds(w * per, per)], didx_v)

    pltpu.sync_copy(ones_hbm, ones_v)
    for q in range(n_sub):
      @pl.when(r0 + q * SUB < n)
      def _init():
        pltpu.sync_copy(
            ones_v.at[pl.ds(0, SUB)], acc_sh.at[pl.ds(r0 + q * SUB, SUB)])
    plsc.subcore_barrier()

    for j in range(per):
      pltpu.sync_copy(
          ones_v.at[pl.ds(0, CH)], acc_sh.at[didx_v.at[j]], add=True)

    plsc.subcore_barrier()
    for q in range(n_sub):
      @pl.when(r0 + q * SUB < n)
      def _out():
        pltpu.sync_copy(
            acc_sh.at[pl.ds(r0 + q * SUB, SUB)],
            out_hbm.at[c, pl.ds(r0 + q * SUB, SUB)])

  return deg_kernel


NBUF = 2


def _make_scatter_kernel(n, e_pad, d):
  full = e_pad // CH
  per = full // NW
  assert per * NW == full and per % 8 == 0 and per % NBUF == 0
  n_sub = STRIPE // SUB
  mesh = _sc_mesh()

  @functools.partial(
      pl.kernel,
      out_type=jax.ShapeDtypeStruct((NC, n, d), jnp.float32),
      mesh=mesh,
      scratch_types=[
          pltpu.VMEM((1, CH), jnp.int32),
          pltpu.VMEM((1, CH), jnp.int32),
          pltpu.VMEM((CH, d), jnp.float32),
          pltpu.VMEM_SHARED((n + JUNK, d), jnp.float32),
      ],
  )
  def scatter_kernel(src_hbm, dst_hbm, y_hbm, zeros_hbm, out_hbm,
                     sidx_v, didx_v, rows_v, acc_sh):
    c = lax.axis_index("c")
    s = lax.axis_index("s")
    w = s * NC + c
    r0 = s * STRIPE

    pltpu.sync_copy(zeros_hbm, rows_v.at[pl.ds(0, SUB)])
    for q in range(n_sub):
      @pl.when(r0 + q * SUB < n)
      def _init():
        pltpu.sync_copy(
            rows_v.at[pl.ds(0, SUB)],
            acc_sh.at[pl.ds(r0 + q * SUB, SUB)])
    plsc.subcore_barrier()

    def body(j, carry):
      base = (w * per + j) * CH
      pltpu.sync_copy(src_hbm.at[pl.ds(base, CH)], sidx_v.at[0])
      pltpu.sync_copy(dst_hbm.at[pl.ds(base, CH)], didx_v.at[0])
      pltpu.sync_copy(y_hbm.at[sidx_v.at[0]], rows_v)
      pltpu.sync_copy(rows_v, acc_sh.at[didx_v.at[0]], add=True)
      return carry

    lax.fori_loop(0, per, body, 0)

    plsc.subcore_barrier()
    for q in range(n_sub):
      @pl.when(r0 + q * SUB < n)
      def _out():
        pltpu.sync_copy(
            acc_sh.at[pl.ds(r0 + q * SUB, SUB)],
            out_hbm.at[c, pl.ds(r0 + q * SUB, SUB)])

  return scatter_kernel


ROW_BLK = 1000


def _tc_first(dcol_ref, x_ref, w_ref, y_ref):
  dinv = lax.rsqrt(dcol_ref[...])
  xw = jnp.dot(x_ref[...], w_ref[...], preferred_element_type=jnp.float32)
  y_ref[...] = xw * dinv


def _tc_mid(dcol_ref, s_ref, y_ref, b_ref, w_ref, o_ref):
  dinv = lax.rsqrt(dcol_ref[...])
  tot = s_ref[0] + s_ref[1] + y_ref[...]
  h = jnp.maximum(tot * dinv + b_ref[...], 0.0)
  o_ref[...] = jnp.dot(
      h, w_ref[...], preferred_element_type=jnp.float32) * dinv


def _tc_last(dcol_ref, s_ref, y_ref, b_ref, w_ref, bl_ref, o_ref):
  dinv = lax.rsqrt(dcol_ref[...])
  tot = s_ref[0] + s_ref[1] + y_ref[...]
  h = jnp.maximum(tot * dinv + b_ref[...], 0.0)
  o_ref[...] = jnp.dot(
      h, w_ref[...], preferred_element_type=jnp.float32) + bl_ref[...]


def _dcol_spec():
  return pl.BlockSpec((ROW_BLK, 1), lambda i: (i, 0))


def _row_spec(d):
  return pl.BlockSpec((ROW_BLK, d), lambda i: (i, 0))


def _full_spec(shape):
  nd = len(shape)
  return pl.BlockSpec(shape, lambda i: (0,) * nd)


def _tc1(dcol, x, w1, n, d):
  return pl.pallas_call(
      _tc_first,
      grid=(n // ROW_BLK,),
      in_specs=[_dcol_spec(), _row_spec(d), _full_spec((d, d))],
      out_specs=_row_spec(d),
      out_shape=jax.ShapeDtypeStruct((n, d), jnp.float32),
  )(dcol, x, w1)


def _tc2(dcol, s1, y1, b1r, w2, n, d):
  return pl.pallas_call(
      _tc_mid,
      grid=(n // ROW_BLK,),
      in_specs=[
          _dcol_spec(),
          pl.BlockSpec((NC, ROW_BLK, d), lambda i: (0, i, 0)),
          _row_spec(d),
          _full_spec((1, d)),
          _full_spec((d, d)),
      ],
      out_specs=_row_spec(d),
      out_shape=jax.ShapeDtypeStruct((n, d), jnp.float32),
  )(dcol, s1, y1, b1r, w2)


def _tc3(dcol, s2, y2, b2r, wl, blr, n, d):
  return pl.pallas_call(
      _tc_last,
      grid=(n // ROW_BLK,),
      in_specs=[
          _dcol_spec(),
          pl.BlockSpec((NC, ROW_BLK, d), lambda i: (0, i, 0)),
          _row_spec(d),
          _full_spec((1, d)),
          _full_spec((d, d)),
          _full_spec((1, d)),
      ],
      out_specs=_row_spec(d),
      out_shape=jax.ShapeDtypeStruct((n, d), jnp.float32),
  )(dcol, s2, y2, b2r, wl, blr)


def kernel(x, edge_index, W1, b1, W2, b2, Wl, bl):
  n, d = x.shape
  e = edge_index.shape[1]

  e_pad = e + (-e) % (CH * 8 * NW)
  pad = e_pad - e
  pad_src = jnp.arange(pad, dtype=jnp.int32) % JUNK
  src1 = jnp.concatenate([edge_index[0], pad_src])
  pad_dst = n + jnp.arange(pad, dtype=jnp.int32) % JUNK
  dst1 = jnp.concatenate([edge_index[1], pad_dst])
  dst = dst1.reshape(e_pad // CH, CH)
  ones_col = jnp.ones((CH, DEGW), jnp.float32)
  zeros_blk = jnp.zeros((SUB, d), jnp.float32)
  b1r = b1.reshape(1, d)
  b2r = b2.reshape(1, d)
  blr = bl.reshape(1, d)

  deg_kernel = _make_deg_kernel(n, e_pad)
  scatter_kernel = _make_scatter_kernel(n, e_pad, d)

  deg2 = deg_kernel(dst, ones_col)
  dcol = deg2[0, :, :1] + deg2[1, :, :1] - 1.0

  y1 = _tc1(dcol, x, W1, n, d)
  s1 = scatter_kernel(src1, dst1, y1, zeros_blk)
  y2 = _tc2(dcol, s1, y1, b1r, W2, n, d)
  s2 = scatter_kernel(src1, dst1, y2, zeros_blk)
  return _tc3(dcol, s2, y2, b2r, Wl, blr, n, d)


# speedup vs baseline: 2.7197x; 1.2859x over previous
import functools

import jax
import jax.numpy as jnp
from jax import lax
from jax.experimental import pallas as pl
from jax.experimental.pallas import tpu as pltpu
from jax.experimental.pallas import tpu_sc as plsc

NC = 2
NS = 16
NW = NC * NS
CH = 128
JUNK = 128


def _sc_mesh():
  return plsc.VectorSubcoreMesh(core_axis_name="c", subcore_axis_name="s")


STRIPE = 640
SUB = 80
DEGW = 16


def _make_deg_kernel(n, e_pad):
  full = e_pad // CH
  per = full // NW
  assert per * NW == full and per % 8 == 0
  n_sub = STRIPE // SUB
  mesh = _sc_mesh()

  @functools.partial(
      pl.kernel,
      out_type=jax.ShapeDtypeStruct((NC, n, DEGW), jnp.float32),
      mesh=mesh,
      scratch_types=[
          pltpu.VMEM((per, CH), jnp.int32),
          pltpu.VMEM((CH, DEGW), jnp.float32),
          pltpu.VMEM_SHARED((n + JUNK, DEGW), jnp.float32),
      ],
  )
  def deg_kernel(dst2_hbm, ones_hbm, out_hbm, didx_v, ones_v, acc_sh):
    c = lax.axis_index("c")
    s = lax.axis_index("s")
    w = s * NC + c
    r0 = s * STRIPE

    pltpu.sync_copy(dst2_hbm.at[pl.ds(w * per, per)], didx_v)

    pltpu.sync_copy(ones_hbm, ones_v)
    for q in range(n_sub):
      @pl.when(r0 + q * SUB < n)
      def _init():
        pltpu.sync_copy(
            ones_v.at[pl.ds(0, SUB)], acc_sh.at[pl.ds(r0 + q * SUB, SUB)])
    plsc.subcore_barrier()

    for j in range(per):
      pltpu.sync_copy(
          ones_v.at[pl.ds(0, CH)], acc_sh.at[didx_v.at[j]], add=True)

    plsc.subcore_barrier()
    for q in range(n_sub):
      @pl.when(r0 + q * SUB < n)
      def _out():
        pltpu.sync_copy(
            acc_sh.at[pl.ds(r0 + q * SUB, SUB)],
            out_hbm.at[c, pl.ds(r0 + q * SUB, SUB)])

  return deg_kernel


NBUF = 2


def _make_scatter_kernel(n, e_pad, d):
  full = e_pad // CH
  per = full // NW
  assert per * NW == full and per % 8 == 0 and per % NBUF == 0
  n_sub = STRIPE // SUB
  mesh = _sc_mesh()

  @functools.partial(
      pl.kernel,
      out_type=jax.ShapeDtypeStruct((NC, n, d), jnp.float32),
      mesh=mesh,
      scratch_types=[
          pltpu.VMEM((per, CH), jnp.int32),
          pltpu.VMEM((per, CH), jnp.int32),
          pltpu.VMEM((CH, d), jnp.float32),
          pltpu.VMEM_SHARED((n + JUNK, d), jnp.float32),
      ],
  )
  def scatter_kernel(src2_hbm, dst2_hbm, y_hbm, zeros_hbm, out_hbm,
                     sidx_v, didx_v, rows_v, acc_sh):
    c = lax.axis_index("c")
    s = lax.axis_index("s")
    w = s * NC + c
    r0 = s * STRIPE

    pltpu.sync_copy(src2_hbm.at[pl.ds(w * per, per)], sidx_v)
    pltpu.sync_copy(dst2_hbm.at[pl.ds(w * per, per)], didx_v)

    pltpu.sync_copy(zeros_hbm, rows_v.at[pl.ds(0, SUB)])
    for q in range(n_sub):
      @pl.when(r0 + q * SUB < n)
      def _init():
        pltpu.sync_copy(
            rows_v.at[pl.ds(0, SUB)],
            acc_sh.at[pl.ds(r0 + q * SUB, SUB)])
    plsc.subcore_barrier()

    for j in range(per):
      pltpu.sync_copy(y_hbm.at[sidx_v.at[j]], rows_v)
      pltpu.sync_copy(rows_v, acc_sh.at[didx_v.at[j]], add=True)

    plsc.subcore_barrier()
    for q in range(n_sub):
      @pl.when(r0 + q * SUB < n)
      def _out():
        pltpu.sync_copy(
            acc_sh.at[pl.ds(r0 + q * SUB, SUB)],
            out_hbm.at[c, pl.ds(r0 + q * SUB, SUB)])

  return scatter_kernel


ROW_BLK = 1000


def _tc_first(dcol_ref, x_ref, w_ref, y_ref):
  dinv = lax.rsqrt(dcol_ref[...])
  xw = jnp.dot(x_ref[...], w_ref[...], preferred_element_type=jnp.float32)
  y_ref[...] = xw * dinv


def _tc_mid(dcol_ref, s_ref, y_ref, b_ref, w_ref, o_ref):
  dinv = lax.rsqrt(dcol_ref[...])
  tot = s_ref[0] + s_ref[1] + y_ref[...]
  h = jnp.maximum(tot * dinv + b_ref[...], 0.0)
  o_ref[...] = jnp.dot(
      h, w_ref[...], preferred_element_type=jnp.float32) * dinv


def _tc_last(dcol_ref, s_ref, y_ref, b_ref, w_ref, bl_ref, o_ref):
  dinv = lax.rsqrt(dcol_ref[...])
  tot = s_ref[0] + s_ref[1] + y_ref[...]
  h = jnp.maximum(tot * dinv + b_ref[...], 0.0)
  o_ref[...] = jnp.dot(
      h, w_ref[...], preferred_element_type=jnp.float32) + bl_ref[...]


def _dcol_spec():
  return pl.BlockSpec((ROW_BLK, 1), lambda i: (i, 0))


def _row_spec(d):
  return pl.BlockSpec((ROW_BLK, d), lambda i: (i, 0))


def _full_spec(shape):
  nd = len(shape)
  return pl.BlockSpec(shape, lambda i: (0,) * nd)


def _tc1(dcol, x, w1, n, d):
  return pl.pallas_call(
      _tc_first,
      grid=(n // ROW_BLK,),
      in_specs=[_dcol_spec(), _row_spec(d), _full_spec((d, d))],
      out_specs=_row_spec(d),
      out_shape=jax.ShapeDtypeStruct((n, d), jnp.float32),
  )(dcol, x, w1)


def _tc2(dcol, s1, y1, b1r, w2, n, d):
  return pl.pallas_call(
      _tc_mid,
      grid=(n // ROW_BLK,),
      in_specs=[
          _dcol_spec(),
          pl.BlockSpec((NC, ROW_BLK, d), lambda i: (0, i, 0)),
          _row_spec(d),
          _full_spec((1, d)),
          _full_spec((d, d)),
      ],
      out_specs=_row_spec(d),
      out_shape=jax.ShapeDtypeStruct((n, d), jnp.float32),
  )(dcol, s1, y1, b1r, w2)


def _tc3(dcol, s2, y2, b2r, wl, blr, n, d):
  return pl.pallas_call(
      _tc_last,
      grid=(n // ROW_BLK,),
      in_specs=[
          _dcol_spec(),
          pl.BlockSpec((NC, ROW_BLK, d), lambda i: (0, i, 0)),
          _row_spec(d),
          _full_spec((1, d)),
          _full_spec((d, d)),
          _full_spec((1, d)),
      ],
      out_specs=_row_spec(d),
      out_shape=jax.ShapeDtypeStruct((n, d), jnp.float32),
  )(dcol, s2, y2, b2r, wl, blr)


def kernel(x, edge_index, W1, b1, W2, b2, Wl, bl):
  n, d = x.shape
  e = edge_index.shape[1]

  e_pad = e + (-e) % (CH * 8 * NW)
  pad = e_pad - e
  pad_src = jnp.arange(pad, dtype=jnp.int32) % JUNK
  src1 = jnp.concatenate([edge_index[0], pad_src])
  pad_dst = n + jnp.arange(pad, dtype=jnp.int32) % JUNK
  dst1 = jnp.concatenate([edge_index[1], pad_dst])
  dst = dst1.reshape(e_pad // CH, CH)
  src = src1.reshape(e_pad // CH, CH)
  ones_col = jnp.ones((CH, DEGW), jnp.float32)
  zeros_blk = jnp.zeros((SUB, d), jnp.float32)
  b1r = b1.reshape(1, d)
  b2r = b2.reshape(1, d)
  blr = bl.reshape(1, d)

  deg_kernel = _make_deg_kernel(n, e_pad)
  scatter_kernel = _make_scatter_kernel(n, e_pad, d)

  deg2 = deg_kernel(dst, ones_col)
  dcol = deg2[0, :, :1] + deg2[1, :, :1] - 1.0

  y1 = _tc1(dcol, x, W1, n, d)
  s1 = scatter_kernel(src, dst, y1, zeros_blk)
  y2 = _tc2(dcol, s1, y1, b1r, W2, n, d)
  s2 = scatter_kernel(src, dst, y2, zeros_blk)
  return _tc3(dcol, s2, y2, b2r, Wl, blr, n, d)


# speedup vs baseline: 3.4530x; 1.2696x over previous
import functools

import jax
import jax.numpy as jnp
from jax import lax
from jax.experimental import pallas as pl
from jax.experimental.pallas import tpu as pltpu
from jax.experimental.pallas import tpu_sc as plsc

NC = 2
NS = 16
NW = NC * NS
CH = 128
JUNK = 128


def _sc_mesh():
  return plsc.VectorSubcoreMesh(core_axis_name="c", subcore_axis_name="s")


STRIPE = 640
SUB = 80
DEGW = 16


def _make_deg_kernel(n, e_pad):
  full = e_pad // CH
  per = full // NW
  assert per * NW == full and per % 8 == 0
  n_sub = STRIPE // SUB
  mesh = _sc_mesh()

  @functools.partial(
      pl.kernel,
      out_type=jax.ShapeDtypeStruct((NC, n, DEGW), jnp.float32),
      mesh=mesh,
      scratch_types=[
          pltpu.VMEM((per, CH), jnp.int32),
          pltpu.VMEM((CH, DEGW), jnp.float32),
          pltpu.VMEM_SHARED((n + JUNK, DEGW), jnp.float32),
      ],
  )
  def deg_kernel(dst2_hbm, ones_hbm, out_hbm, didx_v, ones_v, acc_sh):
    c = lax.axis_index("c")
    s = lax.axis_index("s")
    w = s * NC + c
    r0 = s * STRIPE

    pltpu.sync_copy(dst2_hbm.at[pl.ds(w * per, per)], didx_v)

    pltpu.sync_copy(ones_hbm, ones_v)
    for q in range(n_sub):
      @pl.when(r0 + q * SUB < n)
      def _init():
        pltpu.sync_copy(
            ones_v.at[pl.ds(0, SUB)], acc_sh.at[pl.ds(r0 + q * SUB, SUB)])
    plsc.subcore_barrier()

    for j in range(per):
      pltpu.sync_copy(
          ones_v.at[pl.ds(0, CH)], acc_sh.at[didx_v.at[j]], add=True)

    plsc.subcore_barrier()
    for q in range(n_sub):
      @pl.when(r0 + q * SUB < n)
      def _out():
        pltpu.sync_copy(
            acc_sh.at[pl.ds(r0 + q * SUB, SUB)],
            out_hbm.at[c, pl.ds(r0 + q * SUB, SUB)])

  return deg_kernel


NBUF = 2


def _make_scatter_kernel(n, e_pad, d):
  full = e_pad // CH
  per = full // NW
  assert per * NW == full and per % 8 == 0 and per % NBUF == 0
  n_sub = STRIPE // SUB
  mesh = _sc_mesh()

  @functools.partial(
      pl.kernel,
      out_type=jax.ShapeDtypeStruct((NC, n, d), jnp.float32),
      mesh=mesh,
      scratch_types=[
          pltpu.VMEM((per, CH), jnp.int32),
          pltpu.VMEM((1, CH), jnp.int32),
          pltpu.VMEM((CH, d), jnp.float32),
          pltpu.VMEM((CH, d), jnp.float32),
          pltpu.VMEM_SHARED((n + JUNK, d), jnp.float32),
          pltpu.SemaphoreType.DMA,
      ],
  )
  def scatter_kernel(src2_hbm, dst2_hbm, y_hbm, zeros_hbm, out_hbm,
                     sidx_v, didx_v, rows_v, rows2_v, acc_sh, sem):
    c = lax.axis_index("c")
    s = lax.axis_index("s")
    w = s * NC + c
    r0 = s * STRIPE

    pltpu.sync_copy(src2_hbm.at[pl.ds(w * per, per)], sidx_v)

    pltpu.sync_copy(zeros_hbm, rows_v.at[pl.ds(0, SUB)])
    for q in range(n_sub):
      @pl.when(r0 + q * SUB < n)
      def _init():
        pltpu.sync_copy(
            rows_v.at[pl.ds(0, SUB)],
            acc_sh.at[pl.ds(r0 + q * SUB, SUB)])
    plsc.subcore_barrier()

    bufs = (rows_v, rows2_v)
    h = pltpu.async_copy(y_hbm.at[sidx_v.at[0]], bufs[0], sem)
    for j in range(per):
      pltpu.sync_copy(dst2_hbm.at[pl.ds(w * per + j, 1)], didx_v)
      h.wait()
      if j + 1 < per:
        h = pltpu.async_copy(
            y_hbm.at[sidx_v.at[j + 1]], bufs[(j + 1) % 2], sem)
      pltpu.sync_copy(bufs[j % 2], acc_sh.at[didx_v.at[0]], add=True)

    plsc.subcore_barrier()
    for q in range(n_sub):
      @pl.when(r0 + q * SUB < n)
      def _out():
        pltpu.sync_copy(
            acc_sh.at[pl.ds(r0 + q * SUB, SUB)],
            out_hbm.at[c, pl.ds(r0 + q * SUB, SUB)])

  return scatter_kernel


ROW_BLK = 1000


def _tc_first(dcol_ref, x_ref, w_ref, y_ref):
  dinv = lax.rsqrt(dcol_ref[...])
  xw = jnp.dot(x_ref[...], w_ref[...], preferred_element_type=jnp.float32)
  y_ref[...] = xw * dinv


def _tc_mid(dcol_ref, s_ref, y_ref, b_ref, w_ref, o_ref):
  dinv = lax.rsqrt(dcol_ref[...])
  tot = s_ref[0] + s_ref[1] + y_ref[...]
  h = jnp.maximum(tot * dinv + b_ref[...], 0.0)
  o_ref[...] = jnp.dot(
      h, w_ref[...], preferred_element_type=jnp.float32) * dinv


def _tc_last(dcol_ref, s_ref, y_ref, b_ref, w_ref, bl_ref, o_ref):
  dinv = lax.rsqrt(dcol_ref[...])
  tot = s_ref[0] + s_ref[1] + y_ref[...]
  h = jnp.maximum(tot * dinv + b_ref[...], 0.0)
  o_ref[...] = jnp.dot(
      h, w_ref[...], preferred_element_type=jnp.float32) + bl_ref[...]


def _dcol_spec():
  return pl.BlockSpec((ROW_BLK, 1), lambda i: (i, 0))


def _row_spec(d):
  return pl.BlockSpec((ROW_BLK, d), lambda i: (i, 0))


def _full_spec(shape):
  nd = len(shape)
  return pl.BlockSpec(shape, lambda i: (0,) * nd)


def _tc1(dcol, x, w1, n, d):
  return pl.pallas_call(
      _tc_first,
      grid=(n // ROW_BLK,),
      in_specs=[_dcol_spec(), _row_spec(d), _full_spec((d, d))],
      out_specs=_row_spec(d),
      out_shape=jax.ShapeDtypeStruct((n, d), jnp.float32),
  )(dcol, x, w1)


def _tc2(dcol, s1, y1, b1r, w2, n, d):
  return pl.pallas_call(
      _tc_mid,
      grid=(n // ROW_BLK,),
      in_specs=[
          _dcol_spec(),
          pl.BlockSpec((NC, ROW_BLK, d), lambda i: (0, i, 0)),
          _row_spec(d),
          _full_spec((1, d)),
          _full_spec((d, d)),
      ],
      out_specs=_row_spec(d),
      out_shape=jax.ShapeDtypeStruct((n, d), jnp.float32),
  )(dcol, s1, y1, b1r, w2)


def _tc3(dcol, s2, y2, b2r, wl, blr, n, d):
  return pl.pallas_call(
      _tc_last,
      grid=(n // ROW_BLK,),
      in_specs=[
          _dcol_spec(),
          pl.BlockSpec((NC, ROW_BLK, d), lambda i: (0, i, 0)),
          _row_spec(d),
          _full_spec((1, d)),
          _full_spec((d, d)),
          _full_spec((1, d)),
      ],
      out_specs=_row_spec(d),
      out_shape=jax.ShapeDtypeStruct((n, d), jnp.float32),
  )(dcol, s2, y2, b2r, wl, blr)


def kernel(x, edge_index, W1, b1, W2, b2, Wl, bl):
  n, d = x.shape
  e = edge_index.shape[1]

  e_pad = e + (-e) % (CH * 8 * NW)
  pad = e_pad - e
  pad_src = jnp.arange(pad, dtype=jnp.int32) % JUNK
  src1 = jnp.concatenate([edge_index[0], pad_src])
  pad_dst = n + jnp.arange(pad, dtype=jnp.int32) % JUNK
  dst1 = jnp.concatenate([edge_index[1], pad_dst])
  dst = dst1.reshape(e_pad // CH, CH)
  src = src1.reshape(e_pad // CH, CH)
  ones_col = jnp.ones((CH, DEGW), jnp.float32)
  zeros_blk = jnp.zeros((SUB, d), jnp.float32)
  b1r = b1.reshape(1, d)
  b2r = b2.reshape(1, d)
  blr = bl.reshape(1, d)

  deg_kernel = _make_deg_kernel(n, e_pad)
  scatter_kernel = _make_scatter_kernel(n, e_pad, d)

  deg2 = deg_kernel(dst, ones_col)
  dcol = deg2[0, :, :1] + deg2[1, :, :1] - 1.0

  y1 = _tc1(dcol, x, W1, n, d)
  s1 = scatter_kernel(src, dst, y1, zeros_blk)
  y2 = _tc2(dcol, s1, y1, b1r, W2, n, d)
  s2 = scatter_kernel(src, dst, y2, zeros_blk)
  return _tc3(dcol, s2, y2, b2r, Wl, blr, n, d)
